# Initial kernel scaffold; baseline (speedup 1.0000x reference)
import jax
import jax.numpy as jnp
from jax.experimental import pallas as pl


def kernel(x, edge_index, batch, W_rel, b_rel, W_root, W_lin):
    raise NotImplementedError("write your pallas kernel here")


# speedup vs baseline: 6.0995x; 6.0995x over previous
import functools

import jax
import jax.numpy as jnp
from jax import lax
from jax.experimental import pallas as pl
from jax.experimental.pallas import tpu as pltpu
from jax.experimental.pallas import tpu_sc as plsc

N = 10000
D = 128
E = 320000
K = 3

NC = 2
NS = 16
NW = NC * NS
ECHUNK = 128
NCHUNKS = E // ECHUNK
ZCHUNK = 80
NZCHUNKS = N // ZCHUNK


def _sc_aggregate_body(h_hbm, src_hbm, dst_hbm, out_hbm,
                       src_v, dst_v, rows_v, accum_sh, sem):
    cid = lax.axis_index("c")
    sid = lax.axis_index("s")
    wid = sid * NC + cid

    zeros16 = jnp.zeros((16,), jnp.float32)

    def zrow(i, _):
        def zcol(j, _):
            rows_v[i, pl.ds(j * 16, 16)] = zeros16
            return 0
        return lax.fori_loop(0, D // 16, zcol, 0)

    lax.fori_loop(0, ECHUNK, zrow, 0)

    nz = (NZCHUNKS - sid + NS - 1) // NS

    def zstripe(j, _):
        r0 = (sid + j * NS) * ZCHUNK
        pltpu.sync_copy(rows_v.at[pl.ds(0, ZCHUNK)],
                        accum_sh.at[pl.ds(r0, ZCHUNK)])
        return 0

    lax.fori_loop(0, nz, zstripe, 0)
    plsc.subcore_barrier()

    nt = (NCHUNKS - wid + NW - 1) // NW

    def chunk_body(i, _):
        off = (wid + i * NW) * ECHUNK
        pltpu.sync_copy(src_hbm.at[pl.ds(off, ECHUNK)], src_v)
        pltpu.sync_copy(dst_hbm.at[pl.ds(off, ECHUNK)], dst_v)
        pltpu.async_copy(h_hbm.at[src_v], rows_v, sem).wait()
        pltpu.sync_copy(rows_v, accum_sh.at[dst_v], add=True)
        return 0

    lax.fori_loop(0, nt, chunk_body, 0)
    plsc.subcore_barrier()

    def wstripe(j, _):
        r0 = (sid + j * NS) * ZCHUNK
        pltpu.sync_copy(accum_sh.at[pl.ds(r0, ZCHUNK)],
                        out_hbm.at[cid, pl.ds(r0, ZCHUNK)])
        return 0

    lax.fori_loop(0, nz, wstripe, 0)


_sc_aggregate = functools.partial(
    pl.kernel,
    out_type=jax.ShapeDtypeStruct((NC, N, D), jnp.float32),
    mesh=plsc.VectorSubcoreMesh(core_axis_name="c", subcore_axis_name="s",
                                num_cores=NC, num_subcores=NS),
    scratch_types=[
        pltpu.VMEM((ECHUNK,), jnp.int32),
        pltpu.VMEM((ECHUNK,), jnp.int32),
        pltpu.VMEM((ECHUNK, D), jnp.float32),
        pltpu.VMEM_SHARED((N, D), jnp.float32),
        pltpu.SemaphoreType.DMA,
    ],
)(_sc_aggregate_body)


RBLK = 400


def _tc_update_body(p_ref, h_ref, wrel_ref, brel_ref, wroot_ref, out_ref):
    aggr = p_ref[0] + p_ref[1]
    t = lax.dot_general(aggr, wrel_ref[...], (((1,), (1,)), ((), ())),
                        preferred_element_type=jnp.float32)
    t += lax.dot_general(h_ref[...], wroot_ref[...], (((1,), (1,)), ((), ())),
                         preferred_element_type=jnp.float32)
    out_ref[...] = jnp.maximum(t + brel_ref[...], 0.0)


def _tc_final_body(p_ref, h_ref, wrel_ref, brel_ref, wroot_ref, wlin_ref,
                   out_ref):
    aggr = p_ref[0] + p_ref[1]
    t = lax.dot_general(aggr, wrel_ref[...], (((1,), (1,)), ((), ())),
                        preferred_element_type=jnp.float32)
    t += lax.dot_general(h_ref[...], wroot_ref[...], (((1,), (1,)), ((), ())),
                         preferred_element_type=jnp.float32)
    hnew = jnp.maximum(t + brel_ref[...], 0.0)
    out_ref[...] = lax.dot_general(hnew, wlin_ref[...], (((1,), (1,)), ((), ())),
                                   preferred_element_type=jnp.float32)


def _tc_update(p, h, W_rel, b_rel2, W_root):
    return pl.pallas_call(
        _tc_update_body,
        grid=(N // RBLK,),
        in_specs=[
            pl.BlockSpec((NC, RBLK, D), lambda i: (0, i, 0)),
            pl.BlockSpec((RBLK, D), lambda i: (i, 0)),
            pl.BlockSpec((D, D), lambda i: (0, 0)),
            pl.BlockSpec((1, D), lambda i: (0, 0)),
            pl.BlockSpec((D, D), lambda i: (0, 0)),
        ],
        out_specs=pl.BlockSpec((RBLK, D), lambda i: (i, 0)),
        out_shape=jax.ShapeDtypeStruct((N, D), jnp.float32),
    )(p, h, W_rel, b_rel2, W_root)


def _tc_final(p, h, W_rel, b_rel2, W_root, W_lin):
    return pl.pallas_call(
        _tc_final_body,
        grid=(N // RBLK,),
        in_specs=[
            pl.BlockSpec((NC, RBLK, D), lambda i: (0, i, 0)),
            pl.BlockSpec((RBLK, D), lambda i: (i, 0)),
            pl.BlockSpec((D, D), lambda i: (0, 0)),
            pl.BlockSpec((1, D), lambda i: (0, 0)),
            pl.BlockSpec((D, D), lambda i: (0, 0)),
            pl.BlockSpec((1, D), lambda i: (0, 0)),
        ],
        out_specs=pl.BlockSpec((RBLK, 1), lambda i: (i, 0)),
        out_shape=jax.ShapeDtypeStruct((N, 1), jnp.float32),
    )(p, h, W_rel, b_rel2, W_root, W_lin)


def kernel(x, edge_index, batch, W_rel, b_rel, W_root, W_lin):
    src = edge_index[0]
    dst = edge_index[1]
    b_rel2 = b_rel.reshape(1, D)
    h = x
    for k in range(K):
        p = _sc_aggregate(h, src, dst)
        if k < K - 1:
            h = _tc_update(p, h, W_rel, b_rel2, W_root)
        else:
            return _tc_final(p, h, W_rel, b_rel2, W_root, W_lin)


# speedup vs baseline: 13.7737x; 2.2582x over previous
import functools

import jax
import jax.numpy as jnp
from jax import lax
from jax.experimental import pallas as pl
from jax.experimental.pallas import tpu as pltpu
from jax.experimental.pallas import tpu_sc as plsc

N = 10000
D = 128
E = 320000
K = 3

NC = 2
NS = 16
NW = NC * NS
ECHUNK = 128
NCHUNKS = E // ECHUNK
ZCHUNK = 80
NZCHUNKS = N // ZCHUNK
EPW = E // NW
NFULL = EPW // ECHUNK
REM = EPW - NFULL * ECHUNK


def _sc_aggregate_body(h_hbm, src_hbm, dst_hbm, out_hbm,
                       src_big, dstb0, dstb1, dst_rem,
                       rows0, rows1, rows_rem, accum_sh,
                       sem0, sem1, semd0, semd1):
    cid = lax.axis_index("c")
    sid = lax.axis_index("s")
    wid = sid * NC + cid
    ebase = wid * EPW

    zeros16 = jnp.zeros((16,), jnp.float32)

    def zrow(i, _):
        def zcol(j, _):
            rows0[i, pl.ds(j * 16, 16)] = zeros16
            return 0
        return lax.fori_loop(0, D // 16, zcol, 0)

    lax.fori_loop(0, ZCHUNK, zrow, 0)

    pltpu.sync_copy(src_hbm.at[pl.ds(ebase, EPW)], src_big)

    nz = (NZCHUNKS - sid + NS - 1) // NS

    def zstripe(j, _):
        r0 = (sid + j * NS) * ZCHUNK
        pltpu.sync_copy(rows0.at[pl.ds(0, ZCHUNK)],
                        accum_sh.at[pl.ds(r0, ZCHUNK)])
        return 0

    lax.fori_loop(0, nz, zstripe, 0)
    plsc.subcore_barrier()

    def start_gather(j, rows, sem):
        pltpu.async_copy(
            h_hbm.at[src_big.at[pl.ds(j * ECHUNK, ECHUNK)]], rows, sem)

    def wait_gather(j, rows, sem):
        pltpu.make_async_copy(
            h_hbm.at[src_big.at[pl.ds(j * ECHUNK, ECHUNK)]], rows, sem).wait()

    def start_dst(j, buf, semd):
        pltpu.async_copy(dst_hbm.at[pl.ds(ebase + j * ECHUNK, ECHUNK)],
                         buf, semd)

    def wait_dst(j, buf, semd):
        pltpu.make_async_copy(dst_hbm.at[pl.ds(ebase + j * ECHUNK, ECHUNK)],
                              buf, semd).wait()

    start_gather(0, rows0, sem0)
    start_dst(0, dstb0, semd0)
    start_gather(1, rows1, sem1)
    start_dst(1, dstb1, semd1)

    def pair_body(i, _):
        j0 = 2 * i
        wait_gather(j0, rows0, sem0)
        wait_dst(j0, dstb0, semd0)

        @pl.when(i < (NFULL // 2) - 1)
        def _():
            start_gather(j0 + 2, rows0, sem0)
            start_dst(j0 + 2, dstb0, semd0)

        pltpu.sync_copy(rows0, accum_sh.at[dstb0], add=True)

        wait_gather(j0 + 1, rows1, sem1)
        wait_dst(j0 + 1, dstb1, semd1)

        @pl.when(i < (NFULL // 2) - 1)
        def _():
            start_gather(j0 + 3, rows1, sem1)
            start_dst(j0 + 3, dstb1, semd1)

        pltpu.sync_copy(rows1, accum_sh.at[dstb1], add=True)
        return 0

    lax.fori_loop(0, NFULL // 2, pair_body, 0)

    pltpu.async_copy(
        h_hbm.at[src_big.at[pl.ds(NFULL * ECHUNK, REM)]], rows_rem, sem0)
    pltpu.sync_copy(dst_hbm.at[pl.ds(ebase + NFULL * ECHUNK, REM)], dst_rem)
    pltpu.make_async_copy(
        h_hbm.at[src_big.at[pl.ds(NFULL * ECHUNK, REM)]], rows_rem, sem0).wait()
    pltpu.sync_copy(rows_rem, accum_sh.at[dst_rem], add=True)

    plsc.subcore_barrier()

    def wstripe(j, _):
        r0 = (sid + j * NS) * ZCHUNK
        pltpu.sync_copy(accum_sh.at[pl.ds(r0, ZCHUNK)],
                        out_hbm.at[cid, pl.ds(r0, ZCHUNK)])
        return 0

    lax.fori_loop(0, nz, wstripe, 0)


_sc_aggregate = functools.partial(
    pl.kernel,
    out_type=jax.ShapeDtypeStruct((NC, N, D), jnp.float32),
    mesh=plsc.VectorSubcoreMesh(core_axis_name="c", subcore_axis_name="s",
                                num_cores=NC, num_subcores=NS),
    scratch_types=[
        pltpu.VMEM((EPW,), jnp.int32),
        pltpu.VMEM((ECHUNK,), jnp.int32),
        pltpu.VMEM((ECHUNK,), jnp.int32),
        pltpu.VMEM((REM,), jnp.int32),
        pltpu.VMEM((ECHUNK, D), jnp.float32),
        pltpu.VMEM((ECHUNK, D), jnp.float32),
        pltpu.VMEM((REM, D), jnp.float32),
        pltpu.VMEM_SHARED((N, D), jnp.float32),
        pltpu.SemaphoreType.DMA,
        pltpu.SemaphoreType.DMA,
        pltpu.SemaphoreType.DMA,
        pltpu.SemaphoreType.DMA,
    ],
)(_sc_aggregate_body)


RBLK = 400


def _tc_update_body(p_ref, h_ref, wrel_ref, brel_ref, wroot_ref, out_ref):
    aggr = p_ref[0] + p_ref[1]
    t = lax.dot_general(aggr, wrel_ref[...], (((1,), (1,)), ((), ())),
                        preferred_element_type=jnp.float32)
    t += lax.dot_general(h_ref[...], wroot_ref[...], (((1,), (1,)), ((), ())),
                         preferred_element_type=jnp.float32)
    out_ref[...] = jnp.maximum(t + brel_ref[...], 0.0)


def _tc_final_body(p_ref, h_ref, wrel_ref, brel_ref, wroot_ref, wlin_ref,
                   out_ref):
    aggr = p_ref[0] + p_ref[1]
    t = lax.dot_general(aggr, wrel_ref[...], (((1,), (1,)), ((), ())),
                        preferred_element_type=jnp.float32)
    t += lax.dot_general(h_ref[...], wroot_ref[...], (((1,), (1,)), ((), ())),
                         preferred_element_type=jnp.float32)
    hnew = jnp.maximum(t + brel_ref[...], 0.0)
    out_ref[...] = lax.dot_general(hnew, wlin_ref[...], (((1,), (1,)), ((), ())),
                                   preferred_element_type=jnp.float32)


def _tc_update(p, h, W_rel, b_rel2, W_root):
    return pl.pallas_call(
        _tc_update_body,
        grid=(N // RBLK,),
        in_specs=[
            pl.BlockSpec((NC, RBLK, D), lambda i: (0, i, 0)),
            pl.BlockSpec((RBLK, D), lambda i: (i, 0)),
            pl.BlockSpec((D, D), lambda i: (0, 0)),
            pl.BlockSpec((1, D), lambda i: (0, 0)),
            pl.BlockSpec((D, D), lambda i: (0, 0)),
        ],
        out_specs=pl.BlockSpec((RBLK, D), lambda i: (i, 0)),
        out_shape=jax.ShapeDtypeStruct((N, D), jnp.float32),
    )(p, h, W_rel, b_rel2, W_root)


def _tc_final(p, h, W_rel, b_rel2, W_root, W_lin):
    return pl.pallas_call(
        _tc_final_body,
        grid=(N // RBLK,),
        in_specs=[
            pl.BlockSpec((NC, RBLK, D), lambda i: (0, i, 0)),
            pl.BlockSpec((RBLK, D), lambda i: (i, 0)),
            pl.BlockSpec((D, D), lambda i: (0, 0)),
            pl.BlockSpec((1, D), lambda i: (0, 0)),
            pl.BlockSpec((D, D), lambda i: (0, 0)),
            pl.BlockSpec((1, D), lambda i: (0, 0)),
        ],
        out_specs=pl.BlockSpec((RBLK, 1), lambda i: (i, 0)),
        out_shape=jax.ShapeDtypeStruct((N, 1), jnp.float32),
    )(p, h, W_rel, b_rel2, W_root, W_lin)


def kernel(x, edge_index, batch, W_rel, b_rel, W_root, W_lin):
    src = edge_index[0]
    dst = edge_index[1]
    b_rel2 = b_rel.reshape(1, D)
    h = x
    for k in range(K):
        p = _sc_aggregate(h, src, dst)
        if k < K - 1:
            h = _tc_update(p, h, W_rel, b_rel2, W_root)
        else:
            return _tc_final(p, h, W_rel, b_rel2, W_root, W_lin)


# speedup vs baseline: 15.1378x; 1.0990x over previous
import functools

import jax
import jax.numpy as jnp
from jax import lax
from jax.experimental import pallas as pl
from jax.experimental.pallas import tpu as pltpu
from jax.experimental.pallas import tpu_sc as plsc

N = 10000
D = 128
E = 320000
K = 3

NC = 2
NS = 16
NW = NC * NS
ECHUNK = 128
NCHUNKS = E // ECHUNK
ZCHUNK = 80
NZCHUNKS = N // ZCHUNK
EPW = E // NW
NFULL = EPW // ECHUNK
REM = EPW - NFULL * ECHUNK


def _sc_aggregate_body(h_hbm, src_hbm, dst_hbm, out_hbm,
                       src_big, dstb0, dstb1, dst_rem,
                       rows0, rows1, rows_rem, accum_sh,
                       sem0, sem1, semd0, semd1):
    cid = lax.axis_index("c")
    sid = lax.axis_index("s")
    wid = sid * NC + cid
    ebase = wid * EPW

    zeros16 = jnp.zeros((16,), jnp.float32)

    def zrow(i, _):
        def zcol(j, _):
            rows0[i, pl.ds(j * 16, 16)] = zeros16
            return 0
        return lax.fori_loop(0, D // 16, zcol, 0)

    lax.fori_loop(0, ZCHUNK, zrow, 0)

    pltpu.sync_copy(src_hbm.at[pl.ds(ebase, EPW)], src_big)

    nz = (NZCHUNKS - sid + NS - 1) // NS

    def zstripe(j, _):
        r0 = (sid + j * NS) * ZCHUNK
        pltpu.sync_copy(rows0.at[pl.ds(0, ZCHUNK)],
                        accum_sh.at[pl.ds(r0, ZCHUNK)])
        return 0

    lax.fori_loop(0, nz, zstripe, 0)
    plsc.subcore_barrier()

    def start_gather(j, rows, sem):
        pltpu.async_copy(
            h_hbm.at[src_big.at[pl.ds(j * ECHUNK, ECHUNK)]], rows, sem)

    def wait_gather(j, rows, sem):
        pltpu.make_async_copy(
            h_hbm.at[src_big.at[pl.ds(j * ECHUNK, ECHUNK)]], rows, sem).wait()

    def start_dst(j, buf, semd):
        pltpu.async_copy(dst_hbm.at[pl.ds(ebase + j * ECHUNK, ECHUNK)],
                         buf, semd)

    def wait_dst(j, buf, semd):
        pltpu.make_async_copy(dst_hbm.at[pl.ds(ebase + j * ECHUNK, ECHUNK)],
                              buf, semd).wait()

    start_gather(0, rows0, sem0)
    start_dst(0, dstb0, semd0)
    start_gather(1, rows1, sem1)
    start_dst(1, dstb1, semd1)

    def pair_body(i, _):
        j0 = 2 * i
        wait_gather(j0, rows0, sem0)
        wait_dst(j0, dstb0, semd0)

        @pl.when(i < (NFULL // 2) - 1)
        def _():
            start_gather(j0 + 2, rows0, sem0)
            start_dst(j0 + 2, dstb0, semd0)

        pltpu.sync_copy(rows0, accum_sh.at[dstb0], add=True)

        wait_gather(j0 + 1, rows1, sem1)
        wait_dst(j0 + 1, dstb1, semd1)

        @pl.when(i < (NFULL // 2) - 1)
        def _():
            start_gather(j0 + 3, rows1, sem1)
            start_dst(j0 + 3, dstb1, semd1)

        pltpu.sync_copy(rows1, accum_sh.at[dstb1], add=True)
        return 0

    lax.fori_loop(0, NFULL // 2, pair_body, 0)

    pltpu.async_copy(
        h_hbm.at[src_big.at[pl.ds(NFULL * ECHUNK, REM)]], rows_rem, sem0)
    pltpu.sync_copy(dst_hbm.at[pl.ds(ebase + NFULL * ECHUNK, REM)], dst_rem)
    pltpu.make_async_copy(
        h_hbm.at[src_big.at[pl.ds(NFULL * ECHUNK, REM)]], rows_rem, sem0).wait()
    pltpu.sync_copy(rows_rem, accum_sh.at[dst_rem], add=True)

    plsc.subcore_barrier()

    def wstripe(j, _):
        r0 = (sid + j * NS) * ZCHUNK
        pltpu.sync_copy(accum_sh.at[pl.ds(r0, ZCHUNK)],
                        out_hbm.at[cid, pl.ds(r0, ZCHUNK)])
        return 0

    lax.fori_loop(0, nz, wstripe, 0)


_sc_aggregate = functools.partial(
    pl.kernel,
    out_type=jax.ShapeDtypeStruct((NC, N, D), jnp.float32),
    mesh=plsc.VectorSubcoreMesh(core_axis_name="c", subcore_axis_name="s",
                                num_cores=NC, num_subcores=NS),
    scratch_types=[
        pltpu.VMEM((EPW,), jnp.int32),
        pltpu.VMEM((ECHUNK,), jnp.int32),
        pltpu.VMEM((ECHUNK,), jnp.int32),
        pltpu.VMEM((REM,), jnp.int32),
        pltpu.VMEM((ECHUNK, D), jnp.float32),
        pltpu.VMEM((ECHUNK, D), jnp.float32),
        pltpu.VMEM((REM, D), jnp.float32),
        pltpu.VMEM_SHARED((N, D), jnp.float32),
        pltpu.SemaphoreType.DMA,
        pltpu.SemaphoreType.DMA,
        pltpu.SemaphoreType.DMA,
        pltpu.SemaphoreType.DMA,
    ],
)(_sc_aggregate_body)


RBLK = 2000


def _tc_update_body(p_ref, h_ref, wrel_ref, brel_ref, wroot_ref, out_ref):
    aggr = p_ref[0] + p_ref[1]
    t = lax.dot_general(aggr, wrel_ref[...], (((1,), (1,)), ((), ())),
                        preferred_element_type=jnp.float32)
    t += lax.dot_general(h_ref[...], wroot_ref[...], (((1,), (1,)), ((), ())),
                         preferred_element_type=jnp.float32)
    out_ref[...] = jnp.maximum(t + brel_ref[...], 0.0)


def _tc_final_body(p_ref, h_ref, wrel_ref, brel_ref, wroot_ref, wlin_ref,
                   out_ref):
    aggr = p_ref[0] + p_ref[1]
    t = lax.dot_general(aggr, wrel_ref[...], (((1,), (1,)), ((), ())),
                        preferred_element_type=jnp.float32)
    t += lax.dot_general(h_ref[...], wroot_ref[...], (((1,), (1,)), ((), ())),
                         preferred_element_type=jnp.float32)
    hnew = jnp.maximum(t + brel_ref[...], 0.0)
    out_ref[...] = lax.dot_general(hnew, wlin_ref[...], (((1,), (1,)), ((), ())),
                                   preferred_element_type=jnp.float32)


def _tc_update(p, h, W_rel, b_rel2, W_root):
    return pl.pallas_call(
        _tc_update_body,
        grid=(N // RBLK,),
        in_specs=[
            pl.BlockSpec((NC, RBLK, D), lambda i: (0, i, 0)),
            pl.BlockSpec((RBLK, D), lambda i: (i, 0)),
            pl.BlockSpec((D, D), lambda i: (0, 0)),
            pl.BlockSpec((1, D), lambda i: (0, 0)),
            pl.BlockSpec((D, D), lambda i: (0, 0)),
        ],
        out_specs=pl.BlockSpec((RBLK, D), lambda i: (i, 0)),
        out_shape=jax.ShapeDtypeStruct((N, D), jnp.float32),
    )(p, h, W_rel, b_rel2, W_root)


def _tc_final(p, h, W_rel, b_rel2, W_root, W_lin):
    return pl.pallas_call(
        _tc_final_body,
        grid=(N // RBLK,),
        in_specs=[
            pl.BlockSpec((NC, RBLK, D), lambda i: (0, i, 0)),
            pl.BlockSpec((RBLK, D), lambda i: (i, 0)),
            pl.BlockSpec((D, D), lambda i: (0, 0)),
            pl.BlockSpec((1, D), lambda i: (0, 0)),
            pl.BlockSpec((D, D), lambda i: (0, 0)),
            pl.BlockSpec((1, D), lambda i: (0, 0)),
        ],
        out_specs=pl.BlockSpec((RBLK, 1), lambda i: (i, 0)),
        out_shape=jax.ShapeDtypeStruct((N, 1), jnp.float32),
    )(p, h, W_rel, b_rel2, W_root, W_lin)


def kernel(x, edge_index, batch, W_rel, b_rel, W_root, W_lin):
    src = edge_index[0]
    dst = edge_index[1]
    b_rel2 = b_rel.reshape(1, D)
    h = x
    for k in range(K):
        p = _sc_aggregate(h, src, dst)
        if k < K - 1:
            h = _tc_update(p, h, W_rel, b_rel2, W_root)
        else:
            return _tc_final(p, h, W_rel, b_rel2, W_root, W_lin)
